# baseline (speedup 1.0000x reference)
import functools

import jax
import jax.numpy as jnp
from jax import lax
from jax.experimental import pallas as pl
from jax.experimental.pallas import tpu as pltpu
from jax.experimental.pallas import tpu_sc as plsc

NUM_E = 8
TOPK = 2
NGRP = 4
EPG = NUM_E // NGRP
T = 2048
TK = T * TOPK
H = 2048
FFN = 1024
SFFN = 2048

BM = 128
NBLK = TK // BM
MAXS = NBLK + NUM_E

_HI = jax.lax.Precision.HIGHEST


def _routing_body(hid_ref, gate_ref, sid_ref, ws_ref, p0_ref, p1_ref,
                  sblk_ref, sexp_ref, sst_ref, sen_ref):
    x = hid_ref[...]
    gw = gate_ref[...]
    logits = jax.lax.dot_general(
        x, gw, (((1,), (1,)), ((), ())),
        preferred_element_type=jnp.float32, precision=_HI)
    m = jnp.max(logits, axis=1, keepdims=True)
    ex = jnp.exp(logits - m)
    scores = ex / jnp.sum(ex, axis=1, keepdims=True)

    r8 = jax.lax.broadcasted_iota(jnp.int32, (NUM_E, NUM_E), 0)
    c8 = jax.lax.broadcasted_iota(jnp.int32, (NUM_E, NUM_E), 1)
    swap = ((r8 ^ 1) == c8).astype(jnp.float32)
    swapped = jax.lax.dot_general(
        scores, swap, (((1,), (0,)), ((), ())),
        preferred_element_type=jnp.float32, precision=_HI)
    gs = jnp.maximum(scores, swapped)

    lane = jax.lax.broadcasted_iota(jnp.int32, (T, NUM_E), 1)
    gidx = lane // EPG
    big = jnp.int32(1 << 20)
    neg = jnp.float32(-jnp.inf)

    v1 = jnp.max(gs, axis=1, keepdims=True)
    g1 = jnp.min(jnp.where(gs == v1, gidx, big), axis=1, keepdims=True)
    gs2 = jnp.where(gidx == g1, neg, gs)
    v2 = jnp.max(gs2, axis=1, keepdims=True)
    g2 = jnp.min(jnp.where(gs2 == v2, gidx, big), axis=1, keepdims=True)
    gmask = (gidx == g1) | (gidx == g2)

    ms = jnp.where(gmask, scores, 0.0)
    w1 = jnp.max(ms, axis=1, keepdims=True)
    e1 = jnp.min(jnp.where(ms == w1, lane, big), axis=1, keepdims=True)
    ms2 = jnp.where(lane == e1, -1.0, ms)
    w2 = jnp.max(ms2, axis=1, keepdims=True)
    e2 = jnp.min(jnp.where(ms2 == w2, lane, big), axis=1, keepdims=True)

    sel = ((lane == e1) | (lane == e2)).astype(jnp.float32)

    nb, bs = 16, T // 16
    s3 = sel.reshape(nb, bs, NUM_E)
    rb = jax.lax.broadcasted_iota(jnp.int32, (bs, bs), 0)
    cb = jax.lax.broadcasted_iota(jnp.int32, (bs, bs), 1)
    lstrict = (cb < rb).astype(jnp.float32)
    lb = jnp.broadcast_to(lstrict, (nb, bs, bs))
    within = jax.lax.dot_general(
        lb, s3, (((2,), (1,)), ((0,), (0,))),
        preferred_element_type=jnp.float32, precision=_HI)
    tot = jnp.sum(s3, axis=1)
    rn = jax.lax.broadcasted_iota(jnp.int32, (nb, nb), 0)
    cn = jax.lax.broadcasted_iota(jnp.int32, (nb, nb), 1)
    ln = (cn < rn).astype(jnp.float32)
    ex_tot = jax.lax.dot_general(
        ln, tot, (((1,), (0,)), ((), ())),
        preferred_element_type=jnp.float32, precision=_HI)
    cum = (within + ex_tot[:, None, :]).reshape(T, NUM_E)

    counts = jnp.sum(sel, axis=0, keepdims=True)
    u8 = (r8 < c8).astype(jnp.float32)
    offs = jax.lax.dot_general(
        counts, u8, (((1,), (0,)), ((), ())),
        preferred_element_type=jnp.float32, precision=_HI)

    slot = offs + cum
    dest0 = jnp.sum(jnp.where(lane == e1, slot, 0.0), axis=1, keepdims=True)
    dest1 = jnp.sum(jnp.where(lane == e2, slot, 0.0), axis=1, keepdims=True)
    p0_ref[...] = dest0.astype(jnp.int32)
    p1_ref[...] = dest1.astype(jnp.int32)

    d0i = dest0.astype(jnp.int32)
    d1i = dest1.astype(jnp.int32)
    tcol = jax.lax.broadcasted_iota(jnp.int32, (T, 1), 0).astype(jnp.float32)
    vals_a = jnp.concatenate([tcol, w1], axis=1)
    vals_b = jnp.concatenate([tcol, w2], axis=1)
    pch = 1024
    for c in range(TK // pch):
        prow = jax.lax.broadcasted_iota(jnp.int32, (1, pch), 1) + c * pch
        m0 = (d0i == prow).astype(jnp.float32)
        m1 = (d1i == prow).astype(jnp.float32)
        chunk = (
            jax.lax.dot_general(m0, vals_a, (((0,), (0,)), ((), ())),
                                preferred_element_type=jnp.float32,
                                precision=_HI)
            + jax.lax.dot_general(m1, vals_b, (((0,), (0,)), ((), ())),
                                  preferred_element_type=jnp.float32,
                                  precision=_HI))
        sid_ref[pl.ds(c * pch, pch), :] = chunk[:, 0:1].astype(jnp.int32)
        ws_ref[pl.ds(c * pch, pch), :] = chunk[:, 1:2]

    bidx = jax.lax.broadcasted_iota(jnp.int32, (NBLK, NUM_E), 0).astype(jnp.float32) * BM
    start = jnp.clip(offs - bidx, 0.0, float(BM))
    endv = jnp.clip(offs + counts - bidx, 0.0, float(BM))
    ov = start < endv
    ovf = ov.astype(jnp.float32)
    dest_r = jax.lax.dot_general(
        ovf, u8, (((1,), (0,)), ((), ())),
        preferred_element_type=jnp.float32, precision=_HI)
    rowtot = jnp.sum(ovf, axis=1, keepdims=True)
    rnb = jax.lax.broadcasted_iota(jnp.int32, (NBLK, NBLK), 0)
    cnb = jax.lax.broadcasted_iota(jnp.int32, (NBLK, NBLK), 1)
    lnb = (cnb < rnb).astype(jnp.float32)
    row_ex = jax.lax.dot_general(
        lnb, rowtot, (((1,), (0,)), ((), ())),
        preferred_element_type=jnp.float32, precision=_HI)
    dstep = dest_r + row_ex
    nst = jnp.sum(rowtot, axis=0, keepdims=True)

    s3i = jax.lax.broadcasted_iota(jnp.int32, (MAXS, NBLK, NUM_E), 0).astype(jnp.float32)
    cmp = ((s3i == dstep[None, :, :]) & ov[None, :, :]).astype(jnp.float32)
    bval = jax.lax.broadcasted_iota(jnp.int32, (NBLK, NUM_E), 0).astype(jnp.float32)
    eval_ = jax.lax.broadcasted_iota(jnp.int32, (NBLK, NUM_E), 1).astype(jnp.float32)

    def _pick(v):
        return jnp.sum(jnp.sum(cmp * v[None, :, :], axis=2), axis=1,
                       keepdims=True)

    scol = jax.lax.broadcasted_iota(jnp.int32, (MAXS, 1), 0).astype(jnp.float32)
    inr = scol < nst
    sblk_ref[...] = jnp.where(inr, _pick(bval), float(NBLK - 1)).astype(jnp.int32)
    sexp_ref[...] = jnp.where(inr, _pick(eval_), float(NUM_E - 1)).astype(jnp.int32)
    sst_ref[...] = jnp.where(inr, _pick(start), 0.0).astype(jnp.int32)
    sen_ref[...] = jnp.where(inr, _pick(endv), 0.0).astype(jnp.int32)


def _routing(hidden, gate_weight):
    return pl.pallas_call(
        _routing_body,
        out_shape=[
            jax.ShapeDtypeStruct((TK, 1), jnp.int32),
            jax.ShapeDtypeStruct((TK, 1), jnp.float32),
            jax.ShapeDtypeStruct((T, 1), jnp.int32),
            jax.ShapeDtypeStruct((T, 1), jnp.int32),
            jax.ShapeDtypeStruct((MAXS, 1), jnp.int32),
            jax.ShapeDtypeStruct((MAXS, 1), jnp.int32),
            jax.ShapeDtypeStruct((MAXS, 1), jnp.int32),
            jax.ShapeDtypeStruct((MAXS, 1), jnp.int32),
        ],
    )(hidden, gate_weight)


def _sc_gather(table, ids):
    n_rows = ids.shape[0]
    n_w = 32
    per_w = n_rows // n_w
    ch = 32
    n_ch = per_w // ch
    mesh = plsc.VectorSubcoreMesh(core_axis_name="c", subcore_axis_name="s")

    @functools.partial(
        pl.kernel, mesh=mesh,
        out_type=jax.ShapeDtypeStruct((n_rows, H), jnp.float32),
        scratch_types=[
            pltpu.VMEM((ch,), jnp.int32),
            pltpu.VMEM((ch, H), jnp.float32),
            pltpu.SemaphoreType.DMA,
        ],
    )
    def k(table_hbm, idx_hbm, out_hbm, idx_v, rows_v, sem):
        wid = lax.axis_index("s") * 2 + lax.axis_index("c")
        base = wid * per_w
        for i in range(n_ch):
            b = base + i * ch
            pltpu.sync_copy(idx_hbm.at[pl.ds(b, ch)], idx_v)
            pltpu.async_copy(table_hbm.at[idx_v], rows_v, sem).wait()
            pltpu.sync_copy(rows_v, out_hbm.at[pl.ds(b, ch)])

    return k(table, ids)


def _grouped_body(sblk_ref, sexp_ref, sst_ref, sen_ref,
                  x_ref, wg_ref, wu_ref, wd_ref, ws_ref, out_ref):
    s = pl.program_id(0)
    blk = sblk_ref[s]
    prev = sblk_ref[jnp.maximum(s - 1, 0)]
    first = (s == 0) | (blk != prev)

    @pl.when(first)
    def _():
        out_ref[...] = jnp.zeros_like(out_ref)

    sb = sst_ref[s]
    se = sen_ref[s]

    @pl.when(se > sb)
    def _():
        x = x_ref[...]
        g = jax.lax.dot_general(
            x, wg_ref[0], (((1,), (1,)), ((), ())),
            preferred_element_type=jnp.float32)
        u = jax.lax.dot_general(
            x, wu_ref[0], (((1,), (1,)), ((), ())),
            preferred_element_type=jnp.float32)
        a = g * jax.nn.sigmoid(g) * u
        rows = jax.lax.broadcasted_iota(jnp.int32, (BM, 1), 0)
        w = jnp.where((rows >= sb) & (rows < se), ws_ref[...], 0.0)
        a = a * w
        out_ref[...] += jax.lax.dot_general(
            a, wd_ref[0], (((1,), (1,)), ((), ())),
            preferred_element_type=jnp.float32)


def _grouped_matmul(sblk, sexp, sst, sen, xs, Wg, Wu, Wd, ws):
    grid_spec = pltpu.PrefetchScalarGridSpec(
        num_scalar_prefetch=4,
        grid=(MAXS,),
        in_specs=[
            pl.BlockSpec((BM, H), lambda s, b, e, st, en: (b[s], 0)),
            pl.BlockSpec((1, FFN, H), lambda s, b, e, st, en: (e[s], 0, 0)),
            pl.BlockSpec((1, FFN, H), lambda s, b, e, st, en: (e[s], 0, 0)),
            pl.BlockSpec((1, H, FFN), lambda s, b, e, st, en: (e[s], 0, 0)),
            pl.BlockSpec((BM, 1), lambda s, b, e, st, en: (b[s], 0)),
        ],
        out_specs=pl.BlockSpec((BM, H), lambda s, b, e, st, en: (b[s], 0)),
    )
    return pl.pallas_call(
        _grouped_body,
        grid_spec=grid_spec,
        out_shape=jax.ShapeDtypeStruct((TK, H), jnp.float32),
    )(sblk, sexp, sst, sen, xs, Wg, Wu, Wd, ws)


def _shared_body(hid_ref, sg_ref, su_ref, sd_ref, out_ref):
    c = pl.program_id(0)

    @pl.when(c == 0)
    def _():
        out_ref[...] = jnp.zeros_like(out_ref)

    x = hid_ref[...]
    g = jax.lax.dot_general(
        x, sg_ref[...], (((1,), (1,)), ((), ())), preferred_element_type=jnp.float32)
    u = jax.lax.dot_general(
        x, su_ref[...], (((1,), (1,)), ((), ())), preferred_element_type=jnp.float32)
    a = g * jax.nn.sigmoid(g) * u
    out_ref[...] += jax.lax.dot_general(
        a, sd_ref[...], (((1,), (1,)), ((), ())), preferred_element_type=jnp.float32)


def _shared_expert(hidden, Sg, Su, Sd):
    fc = 256
    nc = SFFN // fc
    return pl.pallas_call(
        _shared_body,
        grid=(nc,),
        in_specs=[
            pl.BlockSpec((T, H), lambda c: (0, 0)),
            pl.BlockSpec((fc, H), lambda c: (c, 0)),
            pl.BlockSpec((fc, H), lambda c: (c, 0)),
            pl.BlockSpec((H, fc), lambda c: (0, c)),
        ],
        out_specs=pl.BlockSpec((T, H), lambda c: (0, 0)),
        out_shape=jax.ShapeDtypeStruct((T, H), jnp.float32),
    )(hidden, Sg, Su, Sd)


def _add3_body(a_ref, b_ref, c_ref, out_ref):
    out_ref[...] = a_ref[...] + b_ref[...] + c_ref[...]


def _add3(a, b, c):
    bt = 256
    return pl.pallas_call(
        _add3_body,
        grid=(T // bt,),
        in_specs=[
            pl.BlockSpec((bt, H), lambda i: (i, 0)),
            pl.BlockSpec((bt, H), lambda i: (i, 0)),
            pl.BlockSpec((bt, H), lambda i: (i, 0)),
        ],
        out_specs=pl.BlockSpec((bt, H), lambda i: (i, 0)),
        out_shape=jax.ShapeDtypeStruct((T, H), jnp.float32),
    )(a, b, c)


def kernel(hidden_states, gate_weight, Wg, Wu, Wd, Sg, Su, Sd):
    b, s, h = hidden_states.shape
    hidden = hidden_states.reshape(-1, h)
    sid, ws, p0, p1, sblk, sexp, sst, sen = _routing(hidden, gate_weight)
    xs = _sc_gather(hidden, sid.reshape(TK))
    y = _grouped_matmul(sblk.reshape(MAXS), sexp.reshape(MAXS),
                        sst.reshape(MAXS), sen.reshape(MAXS),
                        xs, Wg, Wu, Wd, ws)
    shared = _shared_expert(hidden, Sg, Su, Sd)
    pos = jnp.concatenate([p0.reshape(T), p1.reshape(T)])
    ab = _sc_gather(y, pos)
    out = _add3(ab[:T], ab[T:], shared)
    return out.reshape(b, s, h)


# speedup vs baseline: 1.1876x; 1.1876x over previous
import functools

import jax
import jax.numpy as jnp
from jax import lax
from jax.experimental import pallas as pl
from jax.experimental.pallas import tpu as pltpu
from jax.experimental.pallas import tpu_sc as plsc

NUM_E = 8
TOPK = 2
NGRP = 4
EPG = NUM_E // NGRP
T = 2048
TK = T * TOPK
H = 2048
FFN = 1024
SFFN = 2048

BM = 128
NBLK = TK // BM
MAXS = NBLK + NUM_E

_HI = jax.lax.Precision.HIGHEST


def _routing_body(hid_ref, gate_ref, p0_ref, p1_ref, w1_ref, w2_ref,
                  sblk_ref, sexp_ref, sst_ref, sen_ref):
    x = hid_ref[...]
    gw = gate_ref[...]
    logits = jax.lax.dot_general(
        x, gw, (((1,), (1,)), ((), ())),
        preferred_element_type=jnp.float32)
    m = jnp.max(logits, axis=1, keepdims=True)
    ex = jnp.exp(logits - m)
    scores = ex / jnp.sum(ex, axis=1, keepdims=True)

    r8 = jax.lax.broadcasted_iota(jnp.int32, (NUM_E, NUM_E), 0)
    c8 = jax.lax.broadcasted_iota(jnp.int32, (NUM_E, NUM_E), 1)
    swap = ((r8 ^ 1) == c8).astype(jnp.float32)
    swapped = jax.lax.dot_general(
        scores, swap, (((1,), (0,)), ((), ())),
        preferred_element_type=jnp.float32, precision=_HI)
    gs = jnp.maximum(scores, swapped)

    lane = jax.lax.broadcasted_iota(jnp.int32, (T, NUM_E), 1)
    gidx = lane // EPG
    big = jnp.int32(1 << 20)
    neg = jnp.float32(-jnp.inf)

    v1 = jnp.max(gs, axis=1, keepdims=True)
    g1 = jnp.min(jnp.where(gs == v1, gidx, big), axis=1, keepdims=True)
    gs2 = jnp.where(gidx == g1, neg, gs)
    v2 = jnp.max(gs2, axis=1, keepdims=True)
    g2 = jnp.min(jnp.where(gs2 == v2, gidx, big), axis=1, keepdims=True)
    gmask = (gidx == g1) | (gidx == g2)

    ms = jnp.where(gmask, scores, 0.0)
    w1 = jnp.max(ms, axis=1, keepdims=True)
    e1 = jnp.min(jnp.where(ms == w1, lane, big), axis=1, keepdims=True)
    ms2 = jnp.where(lane == e1, -1.0, ms)
    w2 = jnp.max(ms2, axis=1, keepdims=True)
    e2 = jnp.min(jnp.where(ms2 == w2, lane, big), axis=1, keepdims=True)

    sel = ((lane == e1) | (lane == e2)).astype(jnp.float32)

    nb, bs = 16, T // 16
    s3 = sel.reshape(nb, bs, NUM_E)
    rb = jax.lax.broadcasted_iota(jnp.int32, (bs, bs), 0)
    cb = jax.lax.broadcasted_iota(jnp.int32, (bs, bs), 1)
    lstrict = (cb < rb).astype(jnp.float32)
    lb = jnp.broadcast_to(lstrict, (nb, bs, bs))
    within = jax.lax.dot_general(
        lb, s3, (((2,), (1,)), ((0,), (0,))),
        preferred_element_type=jnp.float32, precision=_HI)
    tot = jnp.sum(s3, axis=1)
    rn = jax.lax.broadcasted_iota(jnp.int32, (nb, nb), 0)
    cn = jax.lax.broadcasted_iota(jnp.int32, (nb, nb), 1)
    ln = (cn < rn).astype(jnp.float32)
    ex_tot = jax.lax.dot_general(
        ln, tot, (((1,), (0,)), ((), ())),
        preferred_element_type=jnp.float32, precision=_HI)
    cum = (within + ex_tot[:, None, :]).reshape(T, NUM_E)

    counts = jnp.sum(sel, axis=0, keepdims=True)
    u8 = (r8 < c8).astype(jnp.float32)
    offs = jax.lax.dot_general(
        counts, u8, (((1,), (0,)), ((), ())),
        preferred_element_type=jnp.float32, precision=_HI)

    slot = offs + cum
    dest0 = jnp.sum(jnp.where(lane == e1, slot, 0.0), axis=1, keepdims=True)
    dest1 = jnp.sum(jnp.where(lane == e2, slot, 0.0), axis=1, keepdims=True)
    p0_ref[...] = dest0.astype(jnp.int32)
    p1_ref[...] = dest1.astype(jnp.int32)
    w1_ref[...] = w1
    w2_ref[...] = w2

    bidx = jax.lax.broadcasted_iota(jnp.int32, (NBLK, NUM_E), 0).astype(jnp.float32) * BM
    start = jnp.clip(offs - bidx, 0.0, float(BM))
    endv = jnp.clip(offs + counts - bidx, 0.0, float(BM))
    ov = start < endv
    ovf = ov.astype(jnp.float32)
    dest_r = jax.lax.dot_general(
        ovf, u8, (((1,), (0,)), ((), ())),
        preferred_element_type=jnp.float32, precision=_HI)
    rowtot = jnp.sum(ovf, axis=1, keepdims=True)
    rnb = jax.lax.broadcasted_iota(jnp.int32, (NBLK, NBLK), 0)
    cnb = jax.lax.broadcasted_iota(jnp.int32, (NBLK, NBLK), 1)
    lnb = (cnb < rnb).astype(jnp.float32)
    row_ex = jax.lax.dot_general(
        lnb, rowtot, (((1,), (0,)), ((), ())),
        preferred_element_type=jnp.float32, precision=_HI)
    dstep = dest_r + row_ex
    nst = jnp.sum(rowtot, axis=0, keepdims=True)

    s3i = jax.lax.broadcasted_iota(jnp.int32, (MAXS, NBLK, NUM_E), 0).astype(jnp.float32)
    cmp = ((s3i == dstep[None, :, :]) & ov[None, :, :]).astype(jnp.float32)
    bval = jax.lax.broadcasted_iota(jnp.int32, (NBLK, NUM_E), 0).astype(jnp.float32)
    eval_ = jax.lax.broadcasted_iota(jnp.int32, (NBLK, NUM_E), 1).astype(jnp.float32)

    def _pick(v):
        return jnp.sum(jnp.sum(cmp * v[None, :, :], axis=2), axis=1,
                       keepdims=True)

    scol = jax.lax.broadcasted_iota(jnp.int32, (MAXS, 1), 0).astype(jnp.float32)
    inr = scol < nst
    sblk_ref[...] = jnp.where(inr, _pick(bval), float(NBLK - 1)).astype(jnp.int32)
    sexp_ref[...] = jnp.where(inr, _pick(eval_), float(NUM_E - 1)).astype(jnp.int32)
    sst_ref[...] = jnp.where(inr, _pick(start), 0.0).astype(jnp.int32)
    sen_ref[...] = jnp.where(inr, _pick(endv), 0.0).astype(jnp.int32)


def _routing(hidden, gate_weight):
    return pl.pallas_call(
        _routing_body,
        out_shape=[
            jax.ShapeDtypeStruct((T, 1), jnp.int32),
            jax.ShapeDtypeStruct((T, 1), jnp.int32),
            jax.ShapeDtypeStruct((T, 1), jnp.float32),
            jax.ShapeDtypeStruct((T, 1), jnp.float32),
            jax.ShapeDtypeStruct((MAXS, 1), jnp.int32),
            jax.ShapeDtypeStruct((MAXS, 1), jnp.int32),
            jax.ShapeDtypeStruct((MAXS, 1), jnp.int32),
            jax.ShapeDtypeStruct((MAXS, 1), jnp.int32),
        ],
    )(hidden, gate_weight)


def _sc_dispatch(hidden, d0r, d1r):
    n_w = 32
    per_w = T // n_w
    nch = per_w // 16
    mesh = plsc.VectorSubcoreMesh(core_axis_name="c", subcore_axis_name="s")

    @functools.partial(
        pl.kernel, mesh=mesh,
        out_type=jax.ShapeDtypeStruct((TK, H), jnp.float32),
        scratch_types=[
            pltpu.VMEM((nch, 16), jnp.int32),
            pltpu.VMEM((nch, 16), jnp.int32),
            pltpu.VMEM((16, H), jnp.float32),
            pltpu.VMEM((16, H), jnp.float32),
            pltpu.SemaphoreType.DMA,
        ],
    )
    def k(hid_hbm, d0_hbm, d1_hbm, xs_hbm, idx0_v, idx1_v, rba, rbb, xsem):
        wid = lax.axis_index("s") * 2 + lax.axis_index("c")
        base = wid * per_w
        rb = wid * nch
        pltpu.sync_copy(d0_hbm.at[pl.ds(rb, nch)], idx0_v)
        pltpu.sync_copy(d1_hbm.at[pl.ds(rb, nch)], idx1_v)
        bufs = (rba, rbb)
        cps = []
        for c in range(nch):
            buf = bufs[c % 2]
            if len(cps) >= 2:
                cps.pop(0).wait()
                cps.pop(0).wait()
            pltpu.sync_copy(hid_hbm.at[pl.ds(base + c * 16, 16)], buf)
            cps.append(pltpu.async_copy(buf, xs_hbm.at[idx0_v.at[c]], xsem))
            cps.append(pltpu.async_copy(buf, xs_hbm.at[idx1_v.at[c]], xsem))
        for cp in cps:
            cp.wait()

    return k(hidden, d0r, d1r)


def _sc_gather(table, ids):
    n_rows = ids.shape[0]
    n_w = 32
    per_w = n_rows // n_w
    ch = 32
    n_ch = per_w // ch
    mesh = plsc.VectorSubcoreMesh(core_axis_name="c", subcore_axis_name="s")

    @functools.partial(
        pl.kernel, mesh=mesh,
        out_type=jax.ShapeDtypeStruct((n_rows, H), jnp.float32),
        scratch_types=[
            pltpu.VMEM((ch,), jnp.int32),
            pltpu.VMEM((ch, H), jnp.float32),
            pltpu.SemaphoreType.DMA,
        ],
    )
    def k(table_hbm, idx_hbm, out_hbm, idx_v, rows_v, sem):
        wid = lax.axis_index("s") * 2 + lax.axis_index("c")
        base = wid * per_w
        for i in range(n_ch):
            b = base + i * ch
            pltpu.sync_copy(idx_hbm.at[pl.ds(b, ch)], idx_v)
            pltpu.async_copy(table_hbm.at[idx_v], rows_v, sem).wait()
            pltpu.sync_copy(rows_v, out_hbm.at[pl.ds(b, ch)])

    return k(table, ids)


def _grouped_body(sblk_ref, sexp_ref, sst_ref, sen_ref,
                  x_ref, wg_ref, wu_ref, wd_ref, out_ref):
    s = pl.program_id(0)
    blk = sblk_ref[s]
    prev = sblk_ref[jnp.maximum(s - 1, 0)]
    first = (s == 0) | (blk != prev)

    @pl.when(first)
    def _():
        out_ref[...] = jnp.zeros_like(out_ref)

    sb = sst_ref[s]
    se = sen_ref[s]

    @pl.when(se > sb)
    def _():
        x = x_ref[...]
        g = jax.lax.dot_general(
            x, wg_ref[0], (((1,), (1,)), ((), ())),
            preferred_element_type=jnp.float32)
        u = jax.lax.dot_general(
            x, wu_ref[0], (((1,), (1,)), ((), ())),
            preferred_element_type=jnp.float32)
        a = g * jax.nn.sigmoid(g) * u
        rows = jax.lax.broadcasted_iota(jnp.int32, (BM, 1), 0)
        a = jnp.where((rows >= sb) & (rows < se), a, 0.0)
        out_ref[...] += jax.lax.dot_general(
            a, wd_ref[0], (((1,), (1,)), ((), ())),
            preferred_element_type=jnp.float32)


def _grouped_matmul(sblk, sexp, sst, sen, xs, Wg, Wu, Wd):
    grid_spec = pltpu.PrefetchScalarGridSpec(
        num_scalar_prefetch=4,
        grid=(MAXS,),
        in_specs=[
            pl.BlockSpec((BM, H), lambda s, b, e, st, en: (b[s], 0)),
            pl.BlockSpec((1, FFN, H), lambda s, b, e, st, en: (e[s], 0, 0)),
            pl.BlockSpec((1, FFN, H), lambda s, b, e, st, en: (e[s], 0, 0)),
            pl.BlockSpec((1, H, FFN), lambda s, b, e, st, en: (e[s], 0, 0)),
        ],
        out_specs=pl.BlockSpec((BM, H), lambda s, b, e, st, en: (b[s], 0)),
    )
    return pl.pallas_call(
        _grouped_body,
        grid_spec=grid_spec,
        out_shape=jax.ShapeDtypeStruct((TK, H), jnp.float32),
    )(sblk, sexp, sst, sen, xs, Wg, Wu, Wd)


def _shared_body(hid_ref, sg_ref, su_ref, sd_ref, out_ref):
    c = pl.program_id(0)

    @pl.when(c == 0)
    def _():
        out_ref[...] = jnp.zeros_like(out_ref)

    x = hid_ref[...]
    g = jax.lax.dot_general(
        x, sg_ref[...], (((1,), (1,)), ((), ())), preferred_element_type=jnp.float32)
    u = jax.lax.dot_general(
        x, su_ref[...], (((1,), (1,)), ((), ())), preferred_element_type=jnp.float32)
    a = g * jax.nn.sigmoid(g) * u
    out_ref[...] += jax.lax.dot_general(
        a, sd_ref[...], (((1,), (1,)), ((), ())), preferred_element_type=jnp.float32)


def _shared_expert(hidden, Sg, Su, Sd):
    fc = 256
    nc = SFFN // fc
    return pl.pallas_call(
        _shared_body,
        grid=(nc,),
        in_specs=[
            pl.BlockSpec((T, H), lambda c: (0, 0)),
            pl.BlockSpec((fc, H), lambda c: (c, 0)),
            pl.BlockSpec((fc, H), lambda c: (c, 0)),
            pl.BlockSpec((H, fc), lambda c: (0, c)),
        ],
        out_specs=pl.BlockSpec((T, H), lambda c: (0, 0)),
        out_shape=jax.ShapeDtypeStruct((T, H), jnp.float32),
    )(hidden, Sg, Su, Sd)


def _combine_body(a_ref, b_ref, c_ref, w1_ref, w2_ref, out_ref):
    out_ref[...] = (a_ref[...] * w1_ref[...] + b_ref[...] * w2_ref[...]
                    + c_ref[...])


def _combine(a, b, c, w1, w2):
    bt = 256
    return pl.pallas_call(
        _combine_body,
        grid=(T // bt,),
        in_specs=[
            pl.BlockSpec((bt, H), lambda i: (i, 0)),
            pl.BlockSpec((bt, H), lambda i: (i, 0)),
            pl.BlockSpec((bt, H), lambda i: (i, 0)),
            pl.BlockSpec((bt, 1), lambda i: (i, 0)),
            pl.BlockSpec((bt, 1), lambda i: (i, 0)),
        ],
        out_specs=pl.BlockSpec((bt, H), lambda i: (i, 0)),
        out_shape=jax.ShapeDtypeStruct((T, H), jnp.float32),
    )(a, b, c, w1, w2)


def kernel(hidden_states, gate_weight, Wg, Wu, Wd, Sg, Su, Sd):
    b, s, h = hidden_states.shape
    hidden = hidden_states.reshape(-1, h)
    p0, p1, w1, w2, sblk, sexp, sst, sen = _routing(hidden, gate_weight)
    xs = _sc_dispatch(hidden, p0.reshape(T // 16, 16), p1.reshape(T // 16, 16))
    y = _grouped_matmul(sblk.reshape(MAXS), sexp.reshape(MAXS),
                        sst.reshape(MAXS), sen.reshape(MAXS),
                        xs, Wg, Wu, Wd)
    shared = _shared_expert(hidden, Sg, Su, Sd)
    pos = jnp.concatenate([p0.reshape(T), p1.reshape(T)])
    ab = _sc_gather(y, pos)
    out = _combine(ab[:T], ab[T:], shared, w1, w2)
    return out.reshape(b, s, h)
